# baseline (speedup 1.0000x reference)
import functools

import jax
import jax.numpy as jnp
from jax import lax
from jax.experimental import pallas as pl
from jax.experimental.pallas import tpu as pltpu
from jax.experimental.pallas import tpu_sc as plsc

_L = 16


def _splat(x):
    return jnp.full((_L,), x, jnp.int32)


@functools.lru_cache(maxsize=None)
def _make_gather(B, D):
    info = plsc.get_sparse_core_info()
    NC, NS = info.num_cores, info.num_subcores
    NW = NC * NS
    b_w = B // NW
    mesh = plsc.VectorSubcoreMesh(core_axis_name="c", subcore_axis_name="s")
    out_struct = jax.ShapeDtypeStruct((D, B), jnp.float32)

    @functools.partial(
        pl.kernel,
        mesh=mesh,
        compiler_params=pltpu.CompilerParams(use_tc_tiling_on_sc=True,
                                             needs_layout_passes=False),
        out_type=(out_struct, out_struct),
        scratch_types=[
            pltpu.VMEM((b_w,), jnp.int32),
            pltpu.VMEM((b_w,), jnp.int32),
            pltpu.VMEM((2, 8, D, 128), jnp.float32),
            pltpu.VMEM((D, b_w), jnp.float32),
            pltpu.VMEM((D, b_w), jnp.float32),
            pltpu.SemaphoreType.DMA,
            pltpu.SemaphoreType.DMA,
        ],
    )
    def gather(uidx_hbm, iidx_hbm, utab_hbm, itab_hbm, uT_out, iT_out,
               uidx_v, iidx_v, blk_v, uT_v, iT_v, semu, semi):
        wid = lax.axis_index("s") * NC + lax.axis_index("c")
        base = wid * b_w
        pltpu.sync_copy(uidx_hbm.at[pl.ds(base, b_w)], uidx_v)
        pltpu.sync_copy(iidx_hbm.at[pl.ds(base, b_w)], iidx_v)
        lane = lax.iota(jnp.int32, _L)
        n_half = b_w // 8

        def scalar(vec, k):
            return jnp.sum(jnp.where(lane == k, vec, 0))

        def vec_at(idx_v, h):
            return idx_v[pl.ds(lax.shift_right_logical(h, 1) * _L, _L)]

        def issue(tab_hbm, vec, k0, buf, sem):
            for k in range(8):
                r = scalar(vec, k0 + k)
                rb = pl.multiple_of(lax.shift_right_logical(r, 7) * 128, 128)
                pltpu.async_copy(tab_hbm.at[:, pl.ds(rb, 128)],
                                 blk_v.at[buf, k], sem)

        def drain_extract(tab_hbm, outT_v, vec, h, k0, buf, sem):
            for k in range(8):
                pltpu.make_async_copy(tab_hbm.at[:, pl.ds(0, 128)],
                                      blk_v.at[buf, k], sem).wait()
            for k in range(8):
                c = _splat(lax.bitwise_and(scalar(vec, k0 + k), 127))
                j = _splat(h * 8 + k)
                v0 = plsc.load_gather(blk_v, [_splat(buf), _splat(k), lane, c])
                v1 = plsc.load_gather(blk_v,
                                      [_splat(buf), _splat(k), lane + _L, c])
                plsc.store_scatter(outT_v, [lane, j], v0)
                plsc.store_scatter(outT_v, [lane + _L, j], v1)

        def half_body(h, carry):
            k0 = lax.bitwise_and(h, 1) * 8
            vu = vec_at(uidx_v, h)
            vi = vec_at(iidx_v, h)
            drain_extract(utab_hbm, uT_v, vu, h, k0, 0, semu)

            @pl.when(h < n_half - 1)
            def _():
                kn = lax.bitwise_and(h + 1, 1) * 8
                issue(utab_hbm, vec_at(uidx_v, h + 1), kn, 0, semu)

            drain_extract(itab_hbm, iT_v, vi, h, k0, 1, semi)

            @pl.when(h < n_half - 1)
            def _():
                kn = lax.bitwise_and(h + 1, 1) * 8
                issue(itab_hbm, vec_at(iidx_v, h + 1), kn, 1, semi)

            return carry

        issue(utab_hbm, uidx_v[pl.ds(0, _L)], 0, 0, semu)
        issue(itab_hbm, iidx_v[pl.ds(0, _L)], 0, 1, semi)
        lax.fori_loop(0, n_half, half_body, 0)

        pltpu.sync_copy(uT_v, uT_out.at[:, pl.ds(base, b_w)])
        pltpu.sync_copy(iT_v, iT_out.at[:, pl.ds(base, b_w)])

    return gather


def _mlp_body(uT_ref, iT_ref, w1uT_ref, w1iT_ref, b1_ref, w2T_ref, b2_ref,
              w3_ref, b3_ref, out_ref):
    x = (jnp.dot(w1uT_ref[...], uT_ref[...],
                 preferred_element_type=jnp.float32)
         + jnp.dot(w1iT_ref[...], iT_ref[...],
                   preferred_element_type=jnp.float32)
         + b1_ref[...])
    x = jnp.maximum(x, 0.0)
    x = jnp.dot(w2T_ref[...], x, preferred_element_type=jnp.float32) + b2_ref[...]
    x = jnp.maximum(x, 0.0)
    o = jnp.sum(x * w3_ref[...], axis=0) + b3_ref[0, 0]
    out_ref[...] = 4.0 / (1.0 + jnp.exp(-o)) + 1.0


@functools.lru_cache(maxsize=None)
def _make_mlp(B, D, H1, H2):
    bt = 2048
    return pl.pallas_call(
        _mlp_body,
        grid=(B // bt,),
        in_specs=[
            pl.BlockSpec((D, bt), lambda b: (0, b)),
            pl.BlockSpec((D, bt), lambda b: (0, b)),
            pl.BlockSpec((H1, D), lambda b: (0, 0)),
            pl.BlockSpec((H1, D), lambda b: (0, 0)),
            pl.BlockSpec((H1, 1), lambda b: (0, 0)),
            pl.BlockSpec((H2, H1), lambda b: (0, 0)),
            pl.BlockSpec((H2, 1), lambda b: (0, 0)),
            pl.BlockSpec((H2, 1), lambda b: (0, 0)),
            pl.BlockSpec((1, 1), lambda b: (0, 0)),
        ],
        out_specs=pl.BlockSpec((bt,), lambda b: (b,)),
        out_shape=jax.ShapeDtypeStruct((B,), jnp.float32),
    )


def kernel(user_idx, item_idx, user_table, item_table, W1, b1, W2, b2, W3, b3):
    B = user_idx.shape[0]
    D = user_table.shape[1]
    H1 = W1.shape[1]
    H2 = W2.shape[1]
    uT_emb, iT_emb = _make_gather(B, D)(
        user_idx.astype(jnp.int32), item_idx.astype(jnp.int32),
        user_table.T, item_table.T)
    out = _make_mlp(B, D, H1, H2)(
        uT_emb, iT_emb, W1[:D].T, W1[D:].T, b1.reshape(H1, 1),
        W2.T, b2.reshape(H2, 1), W3, b3.reshape(1, 1))
    return out.reshape(B, 1)


# speedup vs baseline: 1.2330x; 1.2330x over previous
import functools

import jax
import jax.numpy as jnp
from jax import lax
from jax.experimental import pallas as pl
from jax.experimental.pallas import tpu as pltpu
from jax.experimental.pallas import tpu_sc as plsc

_L = 16


def _splat(x):
    return jnp.full((_L,), x, jnp.int32)


@functools.lru_cache(maxsize=None)
def _make_gather(B, D):
    info = plsc.get_sparse_core_info()
    NC, NS = info.num_cores, info.num_subcores
    NW = NC * NS
    b_w = B // NW
    mesh = plsc.VectorSubcoreMesh(core_axis_name="c", subcore_axis_name="s")
    out_struct = jax.ShapeDtypeStruct((D, B), jnp.float32)

    @functools.partial(
        pl.kernel,
        mesh=mesh,
        compiler_params=pltpu.CompilerParams(use_tc_tiling_on_sc=True,
                                             needs_layout_passes=False),
        out_type=(out_struct, out_struct),
        scratch_types=[
            pltpu.VMEM((b_w,), jnp.int32),
            pltpu.VMEM((b_w,), jnp.int32),
            pltpu.VMEM((2, 8, D, 128), jnp.float32),
            pltpu.VMEM((D, b_w), jnp.float32),
            pltpu.VMEM((D, b_w), jnp.float32),
            pltpu.SemaphoreType.DMA,
        ],
    )
    def gather(uidx_hbm, iidx_hbm, utab_hbm, itab_hbm, uT_out, iT_out,
               uidx_v, iidx_v, blk_v, uT_v, iT_v, sem):
        wid = lax.axis_index("s") * NC + lax.axis_index("c")
        base = wid * b_w
        pltpu.sync_copy(uidx_hbm.at[pl.ds(base, b_w)], uidx_v)
        pltpu.sync_copy(iidx_hbm.at[pl.ds(base, b_w)], iidx_v)
        lane = lax.iota(jnp.int32, _L)
        n_pair = b_w // _L

        def scalar(vec, k):
            return jnp.sum(jnp.where(lane == k, vec, 0))

        def issue(tab_hbm, vec, k0, buf):
            for k in range(8):
                r = scalar(vec, k0 + k)
                rb = pl.multiple_of(lax.shift_right_logical(r, 7) * 128, 128)
                pltpu.async_copy(tab_hbm.at[:, pl.ds(rb, 128)],
                                 blk_v.at[buf, k], sem)

        def drain_extract(tab_hbm, outT_v, vec, p, k0, buf):
            for k in range(8):
                pltpu.make_async_copy(tab_hbm.at[:, pl.ds(0, 128)],
                                      blk_v.at[buf, k], sem).wait()
            for k in range(8):
                c = _splat(lax.bitwise_and(scalar(vec, k0 + k), 127))
                j = _splat(p * _L + k0 + k)
                v0 = plsc.load_gather(blk_v, [_splat(buf), _splat(k), lane, c])
                v1 = plsc.load_gather(blk_v,
                                      [_splat(buf), _splat(k), lane + _L, c])
                plsc.store_scatter(outT_v, [lane, j], v0)
                plsc.store_scatter(outT_v, [lane + _L, j], v1)

        for idx_v, tab_hbm, outT_v in ((uidx_v, utab_hbm, uT_v),
                                       (iidx_v, itab_hbm, iT_v)):
            def pair_body(p, carry, idx_v=idx_v, tab_hbm=tab_hbm,
                          outT_v=outT_v):
                vec = idx_v[pl.ds(p * _L, _L)]
                issue(tab_hbm, vec, 8, 1)
                drain_extract(tab_hbm, outT_v, vec, p, 0, 0)

                @pl.when(p < n_pair - 1)
                def _():
                    vecn = idx_v[pl.ds((p + 1) * _L, _L)]
                    issue(tab_hbm, vecn, 0, 0)

                drain_extract(tab_hbm, outT_v, vec, p, 8, 1)
                return carry

            vec0 = idx_v[pl.ds(0, _L)]
            issue(tab_hbm, vec0, 0, 0)
            lax.fori_loop(0, n_pair, pair_body, 0)

        pltpu.sync_copy(uT_v, uT_out.at[:, pl.ds(base, b_w)])
        pltpu.sync_copy(iT_v, iT_out.at[:, pl.ds(base, b_w)])

    return gather


def _mlp_body(uT_ref, iT_ref, w1uT_ref, w1iT_ref, b1_ref, w2T_ref, b2_ref,
              w3_ref, b3_ref, out_ref):
    x = (jnp.dot(w1uT_ref[...], uT_ref[...],
                 preferred_element_type=jnp.float32)
         + jnp.dot(w1iT_ref[...], iT_ref[...],
                   preferred_element_type=jnp.float32)
         + b1_ref[...])
    x = jnp.maximum(x, 0.0)
    x = jnp.dot(w2T_ref[...], x, preferred_element_type=jnp.float32) + b2_ref[...]
    x = jnp.maximum(x, 0.0)
    o = jnp.sum(x * w3_ref[...], axis=0) + b3_ref[0, 0]
    out_ref[...] = 4.0 / (1.0 + jnp.exp(-o)) + 1.0


@functools.lru_cache(maxsize=None)
def _make_mlp(B, D, H1, H2):
    bt = 2048
    return pl.pallas_call(
        _mlp_body,
        grid=(B // bt,),
        in_specs=[
            pl.BlockSpec((D, bt), lambda b: (0, b)),
            pl.BlockSpec((D, bt), lambda b: (0, b)),
            pl.BlockSpec((H1, D), lambda b: (0, 0)),
            pl.BlockSpec((H1, D), lambda b: (0, 0)),
            pl.BlockSpec((H1, 1), lambda b: (0, 0)),
            pl.BlockSpec((H2, H1), lambda b: (0, 0)),
            pl.BlockSpec((H2, 1), lambda b: (0, 0)),
            pl.BlockSpec((H2, 1), lambda b: (0, 0)),
            pl.BlockSpec((1, 1), lambda b: (0, 0)),
        ],
        out_specs=pl.BlockSpec((bt,), lambda b: (b,)),
        out_shape=jax.ShapeDtypeStruct((B,), jnp.float32),
    )


def kernel(user_idx, item_idx, user_table, item_table, W1, b1, W2, b2, W3, b3):
    B = user_idx.shape[0]
    D = user_table.shape[1]
    H1 = W1.shape[1]
    H2 = W2.shape[1]
    uT_emb, iT_emb = _make_gather(B, D)(
        user_idx.astype(jnp.int32), item_idx.astype(jnp.int32),
        user_table.T, item_table.T)
    out = _make_mlp(B, D, H1, H2)(
        uT_emb, iT_emb, W1[:D].T, W1[D:].T, b1.reshape(H1, 1),
        W2.T, b2.reshape(H2, 1), W3, b3.reshape(1, 1))
    return out.reshape(B, 1)
